# initial kernel scaffold (unmeasured)
import jax
import jax.numpy as jnp
from jax import lax
from jax.experimental import pallas as pl
from jax.experimental.pallas import tpu as pltpu

N_DEV = 4
N_TOK = 1024
D_IN = 256
D_OUT = 512
E_LOCAL = 4
CAP = 51
ROWS_PER = N_TOK // N_DEV


def kernel(x, router_W, route_idx, expert_W):
    del router_W

    def body(x_ref, idx_ref, w_ref, out_ref, c_ref, recv_buf, send_sems, recv_sems):
        my = lax.axis_index("i")

        barrier_sem = pltpu.get_barrier_semaphore()
        for o in range(1, N_DEV):
            pl.semaphore_signal(
                barrier_sem, inc=1,
                device_id=((my + o) % N_DEV,),
                device_id_type=pl.DeviceIdType.MESH,
            )

        e = idx_ref[:, :]
        expert_ids = lax.broadcasted_iota(jnp.int32, (N_TOK, 16), 1)
        onehot = (e == expert_ids).astype(jnp.float32)
        row_i = lax.broadcasted_iota(jnp.int32, (N_TOK, N_TOK), 0)
        col_j = lax.broadcasted_iota(jnp.int32, (N_TOK, N_TOK), 1)
        lower = (col_j < row_i).astype(jnp.float32)
        prefix = lax.dot_general(
            lower, onehot, (((1,), (0,)), ((), ())),
            preferred_element_type=jnp.float32,
        )
        pos = jnp.sum(prefix * onehot, axis=1, keepdims=True)
        keep = pos < float(CAP)

        x_bf = x_ref[:, :].astype(jnp.bfloat16)
        acc = jnp.zeros((N_TOK, D_OUT), dtype=jnp.float32)
        for l in range(E_LOCAL):
            e_id = my * E_LOCAL + l
            sel = jnp.logical_and(e == e_id, keep).astype(jnp.bfloat16)
            xm = x_bf * sel
            wl = w_ref[l, :, :].astype(jnp.bfloat16)
            acc = acc + lax.dot_general(
                xm, wl, (((1,), (0,)), ((), ())),
                preferred_element_type=jnp.float32,
            )
        c_ref[:, :] = acc.astype(jnp.bfloat16)

        pl.semaphore_wait(barrier_sem, N_DEV - 1)

        rdmas = []
        for o in range(1, N_DEV):
            p = (my + o) % N_DEV
            rdma = pltpu.make_async_remote_copy(
                src_ref=c_ref.at[pl.ds(p * ROWS_PER, ROWS_PER), :],
                dst_ref=recv_buf.at[o],
                send_sem=send_sems.at[o],
                recv_sem=recv_sems.at[o],
                device_id=(p,),
                device_id_type=pl.DeviceIdType.MESH,
            )
            rdma.start()
            rdmas.append(rdma)

        own = lax.dynamic_slice(acc, (my * ROWS_PER, 0), (ROWS_PER, D_OUT))

        total = own
        for o in range(1, N_DEV):
            rdmas[o - 1].wait_recv()
            total = total + recv_buf[o, :, :].astype(jnp.float32)
        out_ref[:, :] = total

        for r in rdmas:
            r.wait_send()

    return pl.pallas_call(
        body,
        out_shape=jax.ShapeDtypeStruct((ROWS_PER, D_OUT), jnp.float32),
        in_specs=[
            pl.BlockSpec(memory_space=pltpu.VMEM),
            pl.BlockSpec(memory_space=pltpu.VMEM),
            pl.BlockSpec(memory_space=pltpu.VMEM),
        ],
        out_specs=pl.BlockSpec(memory_space=pltpu.VMEM),
        scratch_shapes=[
            pltpu.VMEM((N_TOK, D_OUT), jnp.bfloat16),
            pltpu.VMEM((N_DEV, ROWS_PER, D_OUT), jnp.bfloat16),
            pltpu.SemaphoreType.DMA((N_DEV,)),
            pltpu.SemaphoreType.DMA((N_DEV,)),
        ],
        compiler_params=pltpu.CompilerParams(collective_id=0),
    )(x, route_idx, expert_W)


# baseline (device time: 17423 ns/iter reference)
import jax
import jax.numpy as jnp
from jax import lax
from jax.experimental import pallas as pl
from jax.experimental.pallas import tpu as pltpu

N_DEV = 4
N_TOK = 1024
D_IN = 256
D_OUT = 512
E_LOCAL = 4
CAP = 51
ROWS_PER = N_TOK // N_DEV


def kernel(x, router_W, route_idx, expert_W):
    del router_W

    def body(x_ref, idx_ref, w_ref, out_ref, c_ref, recv_buf, send_sems, recv_sems):
        my = lax.axis_index("i")

        barrier_sem = pltpu.get_barrier_semaphore()
        for o in range(1, N_DEV):
            pl.semaphore_signal(
                barrier_sem, inc=1,
                device_id=((my + o) % N_DEV,),
                device_id_type=pl.DeviceIdType.MESH,
            )

        e = idx_ref[:, :]
        expert_ids = lax.broadcasted_iota(jnp.int32, (N_TOK, 16), 1)
        onehot = (e == expert_ids).astype(jnp.float32)
        row_i = lax.broadcasted_iota(jnp.int32, (N_TOK, N_TOK), 0)
        col_j = lax.broadcasted_iota(jnp.int32, (N_TOK, N_TOK), 1)
        lower = (col_j < row_i).astype(jnp.float32)
        prefix = lax.dot_general(
            lower, onehot, (((1,), (0,)), ((), ())),
            preferred_element_type=jnp.float32,
        )
        pos = jnp.sum(prefix * onehot, axis=1, keepdims=True)
        keep = pos < float(CAP)

        x_bf = x_ref[:, :].astype(jnp.bfloat16)
        acc = jnp.zeros((N_TOK, D_OUT), dtype=jnp.float32)
        for l in range(E_LOCAL):
            e_id = my * E_LOCAL + l
            sel = jnp.logical_and(e == e_id, keep).astype(jnp.bfloat16)
            xm = x_bf * sel
            wl = w_ref[l, :, :].astype(jnp.bfloat16)
            acc = acc + lax.dot_general(
                xm, wl, (((1,), (0,)), ((), ())),
                preferred_element_type=jnp.float32,
            )
        c_ref[:, :] = acc.astype(jnp.bfloat16)

        pl.semaphore_wait(barrier_sem, N_DEV - 1)

        rdmas = []
        for o in range(1, N_DEV):
            p = (my + o) % N_DEV
            rdma = pltpu.make_async_remote_copy(
                src_ref=c_ref.at[pl.ds(p * ROWS_PER, ROWS_PER), :],
                dst_ref=recv_buf.at[o],
                send_sem=send_sems.at[o],
                recv_sem=recv_sems.at[o],
                device_id=(p,),
                device_id_type=pl.DeviceIdType.MESH,
            )
            rdma.start()
            rdmas.append(rdma)

        own = c_ref[pl.ds(my * ROWS_PER, ROWS_PER), :].astype(jnp.float32)

        total = own
        for o in range(1, N_DEV):
            rdmas[o - 1].wait_recv()
            total = total + recv_buf[o, :, :].astype(jnp.float32)
        out_ref[:, :] = total

        for r in rdmas:
            r.wait_send()

    return pl.pallas_call(
        body,
        out_shape=jax.ShapeDtypeStruct((ROWS_PER, D_OUT), jnp.float32),
        in_specs=[
            pl.BlockSpec(memory_space=pltpu.VMEM),
            pl.BlockSpec(memory_space=pltpu.VMEM),
            pl.BlockSpec(memory_space=pltpu.VMEM),
        ],
        out_specs=pl.BlockSpec(memory_space=pltpu.VMEM),
        scratch_shapes=[
            pltpu.VMEM((N_TOK, D_OUT), jnp.bfloat16),
            pltpu.VMEM((N_DEV, ROWS_PER, D_OUT), jnp.bfloat16),
            pltpu.SemaphoreType.DMA((N_DEV,)),
            pltpu.SemaphoreType.DMA((N_DEV,)),
        ],
        compiler_params=pltpu.CompilerParams(collective_id=0),
    )(x, route_idx, expert_W)
